# baseline (speedup 1.0000x reference)
import functools

import jax
import jax.numpy as jnp
from jax import lax
from jax.experimental import pallas as pl
from jax.experimental.pallas import tpu as pltpu
from jax.experimental.pallas import tpu_sc as plsc

_BT = 32768
_D = 64
_E = 1000000
_NW = 32
_ROWS_W = _BT // _NW
_CHUNK = 128
_NCHUNK = _ROWS_W // _CHUNK
_IM = 128
_NUNIT = _E // 128
_LMBDA = 0.01

_COMPACT = pltpu.CompilerParams(
    needs_layout_passes=False, use_tc_tiling_on_sc=True)


def _sc_format(ent_t):
    mesh = plsc.VectorSubcoreMesh(core_axis_name="c", subcore_axis_name="s")

    @functools.partial(
        pl.kernel,
        mesh=mesh,
        compiler_params=_COMPACT,
        out_type=jax.ShapeDtypeStruct((_E // 2, 128), jnp.float32),
        scratch_types=(
            [pltpu.VMEM((_D, 128), jnp.float32)] * 8
            + [pltpu.SemaphoreType.DMA] * 8
        ),
    )
    def k(ent_hbm, out_hbm, *bufs):
        ins = bufs[0:4]
        ots = bufs[4:8]
        sis = bufs[8:12]
        sos = bufs[12:16]
        wid = lax.axis_index("s") * 2 + lax.axis_index("c")
        lane = lax.iota(jnp.int32, 16)
        rhalf = lax.shift_right_logical(lane, 1)
        cpar = (lane & 1) * 64
        rows_v = [cs * 8 + rhalf for cs in range(8)]

        def start_in(k_, b):
            u = wid + _NW * k_

            @pl.when(u < _NUNIT)
            def _():
                pltpu.async_copy(
                    ent_hbm.at[:, pl.ds(u * 128, 128)], ins[b], sis[b])

        def transpose(inb, otb):
            for d in range(_D):
                cv = cpar + d
                for cs in range(8):
                    v = inb[d, pl.ds(cs * 16, 16)]
                    plsc.store_scatter(otb, [rows_v[cs], cv], v)

        for b in range(4):
            start_in(b, b)

        def step(kk, carry):
            for b in range(4):
                k_ = kk * 4 + b
                u = wid + _NW * k_

                @pl.when(u < _NUNIT)
                def _():
                    pltpu.make_async_copy(
                        ent_hbm.at[:, pl.ds(0, 128)], ins[b],
                        sis[b]).wait()

                    @pl.when(k_ >= 4)
                    def _():
                        pltpu.make_async_copy(
                            ots[b], out_hbm.at[pl.ds(0, _D)],
                            sos[b]).wait()

                    transpose(ins[b], ots[b])
                    pltpu.async_copy(
                        ots[b], out_hbm.at[pl.ds(u * _D, _D)], sos[b])
                    start_in(k_ + 4, b)

            return carry

        nstep = (_NUNIT // _NW + 4) // 4 + 1
        lax.fori_loop(0, nstep, step, 0)
        for b in range(4):
            pltpu.make_async_copy(
                ots[b], out_hbm.at[pl.ds(0, _D)], sos[b]).wait()

        @pl.when(wid == _NW - 1)
        def _():
            pltpu.sync_copy(
                ent_hbm.at[:, pl.ds((_E - 128) // 128 * 128, 128)], ins[0])
            for d in range(_D):
                for cs in range(8):
                    v = ins[0][d, pl.ds(cs * 16, 16)]
                    plsc.store_scatter(ots[0], [rows_v[cs], cpar + d], v)
            pltpu.sync_copy(
                ots[0].at[pl.ds(32, 32)],
                out_hbm.at[pl.ds(_E // 2 - 32, 32)])

    return k(ent_t)


def _sc_gather_score(table, rel2, h_idx, t_idx, r_idx):
    mesh = plsc.VectorSubcoreMesh(core_axis_name="c", subcore_axis_name="s")

    @functools.partial(
        pl.kernel,
        mesh=mesh,
        compiler_params=_COMPACT,
        out_type=[
            jax.ShapeDtypeStruct((_BT // _IM, _IM), jnp.float32),
            jax.ShapeDtypeStruct((_NW, 128), jnp.float32),
        ],
        scratch_types=[
            pltpu.VMEM((_CHUNK,), jnp.int32),
            pltpu.VMEM((_CHUNK,), jnp.int32),
            pltpu.VMEM((_CHUNK,), jnp.int32),
            pltpu.VMEM((_CHUNK, 128), jnp.float32),
            pltpu.VMEM((_CHUNK, 128), jnp.float32),
            pltpu.VMEM((_CHUNK, 128), jnp.float32),
            pltpu.VMEM((_CHUNK,), jnp.float32),
            pltpu.VMEM((256,), jnp.float32),
            pltpu.VMEM((128,), jnp.float32),
            pltpu.VMEM((_CHUNK,), jnp.int32),
            pltpu.VMEM((_CHUNK,), jnp.int32),
            pltpu.VMEM((_CHUNK,), jnp.int32),
            pltpu.SemaphoreType.DMA,
        ],
    )
    def k(tab_hbm, rel_hbm, hidx_hbm, tidx_hbm, ridx_hbm,
          scores_out, sumsq_out,
          hpair, tpair, rpair, hbuf, tbuf, rbuf,
          scores_v, pbuf, sq_v, hoffv, toffv, roffv, sem):
        wid = lax.axis_index("s") * 2 + lax.axis_index("c")
        lane = lax.iota(jnp.int32, 16)
        sq = jnp.zeros((16,), jnp.float32)
        for c in range(_NCHUNK):
            base = wid * _ROWS_W + c * _CHUNK
            row0 = base // _IM
            pltpu.sync_copy(hidx_hbm.at[row0], hpair)
            pltpu.sync_copy(tidx_hbm.at[row0], tpair)
            pltpu.sync_copy(ridx_hbm.at[row0], rpair)
            for g in range(_CHUNK // 16):
                seg = pl.ds(g * 16, 16)
                hoffv[seg] = (hpair[seg] & 1) * 64
                toffv[seg] = (tpair[seg] & 1) * 64
                roffv[seg] = (rpair[seg] & 1) * 64
                hpair[seg] = lax.shift_right_logical(hpair[seg], 1)
                tpair[seg] = lax.shift_right_logical(tpair[seg], 1)
                rpair[seg] = lax.shift_right_logical(rpair[seg], 1)
            copies = [
                pltpu.async_copy(tab_hbm.at[hpair], hbuf, sem),
                pltpu.async_copy(tab_hbm.at[tpair], tbuf, sem),
                pltpu.async_copy(rel_hbm.at[rpair], rbuf, sem),
            ]
            for cp in copies:
                cp.wait()

            def outer(bi, sq):
                def rowfn(i, sq):
                    j = bi * 16 + i
                    jv = jnp.full((16,), j, jnp.int32)
                    hm = plsc.load_gather(hoffv, [jv]) > 0
                    tm = plsc.load_gather(toffv, [jv]) > 0
                    rm = plsc.load_gather(roffv, [jv]) > 0
                    hr = hbuf.at[j]
                    tr = tbuf.at[j]
                    rr = rbuf.at[j]
                    p = jnp.zeros((16,), jnp.float32)
                    for g in range(_D // 16):
                        s0 = pl.ds(g * 16, 16)
                        s1 = pl.ds(64 + g * 16, 16)
                        hv = jnp.where(hm, hr[s1], hr[s0])
                        tv = jnp.where(tm, tr[s1], tr[s0])
                        rv = jnp.where(rm, rr[s1], rr[s0])
                        p = p + hv * tv * rv
                        sq = sq + hv * hv + tv * tv + rv * rv
                    pbuf[pl.ds(j * 16, 16)] = p
                    return sq

                sq = lax.fori_loop(0, 16, rowfn, sq)
                acc = jnp.zeros((16,), jnp.float32)
                for j in range(16):
                    acc = acc + plsc.load_gather(
                        pbuf, [(bi * 16 + lane) * 16 + j])
                scores_v[pl.ds(bi * 16, 16)] = acc
                return sq

            sq = lax.fori_loop(0, _CHUNK // 16, outer, sq)
            pltpu.sync_copy(scores_v, scores_out.at[row0])
        for g in range(8):
            sq_v[pl.ds(g * 16, 16)] = jnp.zeros((16,), jnp.float32)
        sq_v[pl.ds(0, 16)] = sq
        pltpu.sync_copy(sq_v, sumsq_out.at[wid])

    return k(table, rel2, h_idx, t_idx, r_idx)


def _tc_finalize(scores2d, sumsq2d):
    nrow = scores2d.shape[0]

    def body(s_ref, q_ref, o_ref):
        s = s_ref[...]
        row = lax.broadcasted_iota(jnp.int32, s.shape, 0)
        y = jnp.where(row < nrow // 2, 1.0, -1.0).astype(jnp.float32)
        z = -s * y
        sp = jnp.maximum(z, 0.0) + jnp.log1p(jnp.exp(-jnp.abs(z)))
        regul = jnp.sum(q_ref[...]) / float(_BT * _D)
        o_ref[0, 0] = jnp.sum(sp) / float(_BT) + _LMBDA * regul

    out = pl.pallas_call(
        body,
        out_shape=jax.ShapeDtypeStruct((1, 1), jnp.float32),
        out_specs=pl.BlockSpec(memory_space=pltpu.SMEM),
    )(scores2d, sumsq2d)
    return out


def kernel(pos_h, pos_r, pos_t, neg_h, neg_r, neg_t, entity_emb, relation_emb):
    h_idx = jnp.concatenate([pos_h, neg_h]).reshape(_BT // _IM, _IM)
    t_idx = jnp.concatenate([pos_t, neg_t]).reshape(_BT // _IM, _IM)
    r_idx = jnp.concatenate([pos_r[:, 0], neg_r[:, 0]]).reshape(_BT // _IM, _IM)
    table = _sc_format(entity_emb.T)
    rel2 = relation_emb.reshape(500, 128)
    scores, sumsq = _sc_gather_score(table, rel2, h_idx, t_idx, r_idx)
    out = _tc_finalize(scores, sumsq)
    return out.reshape(())


# speedup vs baseline: 1.9340x; 1.9340x over previous
import functools

import jax
import jax.numpy as jnp
from jax import lax
from jax.experimental import pallas as pl
from jax.experimental.pallas import tpu as pltpu
from jax.experimental.pallas import tpu_sc as plsc

_BT = 32768
_D = 64
_E = 1000000
_NW = 32
_ROWS_W = _BT // _NW
_CHUNK = 128
_NCHUNK = _ROWS_W // _CHUNK
_IM = 128
_NUNIT = _E // 128
_LMBDA = 0.01

_COMPACT = pltpu.CompilerParams(
    needs_layout_passes=False, use_tc_tiling_on_sc=True)


def _sc_format(ent_t):
    mesh = plsc.VectorSubcoreMesh(core_axis_name="c", subcore_axis_name="s")

    @functools.partial(
        pl.kernel,
        mesh=mesh,
        compiler_params=_COMPACT,
        out_type=jax.ShapeDtypeStruct((_E // 2, 128), jnp.float32),
        scratch_types=(
            [pltpu.VMEM((_D, 128), jnp.float32)] * 8
            + [pltpu.SemaphoreType.DMA] * 8
        ),
    )
    def k(ent_hbm, out_hbm, *bufs):
        ins = bufs[0:4]
        ots = bufs[4:8]
        sis = bufs[8:12]
        sos = bufs[12:16]
        wid = lax.axis_index("s") * 2 + lax.axis_index("c")
        lane = lax.iota(jnp.int32, 16)
        rhalf = lax.shift_right_logical(lane, 1)
        cpar = (lane & 1) * 64
        rows_v = [cs * 8 + rhalf for cs in range(8)]

        def start_in(k_, b):
            u = wid + _NW * k_

            @pl.when(u < _NUNIT)
            def _():
                pltpu.async_copy(
                    ent_hbm.at[:, pl.ds(u * 128, 128)], ins[b], sis[b])

        def transpose(inb, otb):
            for d in range(_D):
                cv = cpar + d
                for cs in range(8):
                    v = inb[d, pl.ds(cs * 16, 16)]
                    plsc.store_scatter(otb, [rows_v[cs], cv], v)

        for b in range(4):
            start_in(b, b)

        def step(kk, carry):
            for b in range(4):
                k_ = kk * 4 + b
                u = wid + _NW * k_

                @pl.when(u < _NUNIT)
                def _():
                    pltpu.make_async_copy(
                        ent_hbm.at[:, pl.ds(0, 128)], ins[b],
                        sis[b]).wait()

                    @pl.when(k_ >= 4)
                    def _():
                        pltpu.make_async_copy(
                            ots[b], out_hbm.at[pl.ds(0, _D)],
                            sos[b]).wait()

                    transpose(ins[b], ots[b])
                    pltpu.async_copy(
                        ots[b], out_hbm.at[pl.ds(u * _D, _D)], sos[b])
                    start_in(k_ + 4, b)

            return carry

        nstep = (_NUNIT // _NW + 4) // 4 + 1
        lax.fori_loop(0, nstep, step, 0)
        for b in range(4):
            pltpu.make_async_copy(
                ots[b], out_hbm.at[pl.ds(0, _D)], sos[b]).wait()

        @pl.when(wid == _NW - 1)
        def _():
            pltpu.sync_copy(
                ent_hbm.at[:, pl.ds((_E - 128) // 128 * 128, 128)], ins[0])
            for d in range(_D):
                for cs in range(8):
                    v = ins[0][d, pl.ds(cs * 16, 16)]
                    plsc.store_scatter(ots[0], [rows_v[cs], cpar + d], v)
            pltpu.sync_copy(
                ots[0].at[pl.ds(32, 32)],
                out_hbm.at[pl.ds(_E // 2 - 32, 32)])

    return k(ent_t)


def _sc_gather_score(table, rel2, h_idx, t_idx, r_idx):
    mesh = plsc.VectorSubcoreMesh(core_axis_name="c", subcore_axis_name="s")

    @functools.partial(
        pl.kernel,
        mesh=mesh,
        compiler_params=_COMPACT,
        out_type=[
            jax.ShapeDtypeStruct((_BT // _IM, _IM), jnp.float32),
            jax.ShapeDtypeStruct((_NW, 128), jnp.float32),
        ],
        scratch_types=[
            pltpu.VMEM((_CHUNK,), jnp.int32),
            pltpu.VMEM((_CHUNK,), jnp.int32),
            pltpu.VMEM((_CHUNK,), jnp.int32),
            pltpu.VMEM((_CHUNK, 128), jnp.float32),
            pltpu.VMEM((_CHUNK, 128), jnp.float32),
            pltpu.VMEM((_CHUNK, 128), jnp.float32),
            pltpu.VMEM((_CHUNK,), jnp.float32),
            pltpu.VMEM((256,), jnp.float32),
            pltpu.VMEM((128,), jnp.float32),
            pltpu.VMEM((_CHUNK,), jnp.int32),
            pltpu.VMEM((_CHUNK,), jnp.int32),
            pltpu.VMEM((_CHUNK,), jnp.int32),
            pltpu.SemaphoreType.DMA,
        ],
    )
    def k(tab_hbm, rel_hbm, hidx_hbm, tidx_hbm, ridx_hbm,
          scores_out, sumsq_out,
          hpair, tpair, rpair, hbuf, tbuf, rbuf,
          scores_v, pbuf, sq_v, hoffv, toffv, roffv, sem):
        wid = lax.axis_index("s") * 2 + lax.axis_index("c")
        lane = lax.iota(jnp.int32, 16)
        sq = jnp.zeros((16,), jnp.float32)
        for c in range(_NCHUNK):
            base = wid * _ROWS_W + c * _CHUNK
            row0 = base // _IM
            pltpu.sync_copy(hidx_hbm.at[row0], hpair)
            pltpu.sync_copy(tidx_hbm.at[row0], tpair)
            pltpu.sync_copy(ridx_hbm.at[row0], rpair)
            for g in range(_CHUNK // 16):
                seg = pl.ds(g * 16, 16)
                hoffv[seg] = (hpair[seg] & 1) * 64
                toffv[seg] = (tpair[seg] & 1) * 64
                roffv[seg] = (rpair[seg] & 1) * 64
                hpair[seg] = lax.shift_right_logical(hpair[seg], 1)
                tpair[seg] = lax.shift_right_logical(tpair[seg], 1)
                rpair[seg] = lax.shift_right_logical(rpair[seg], 1)
            copies = [
                pltpu.async_copy(tab_hbm.at[hpair], hbuf, sem),
                pltpu.async_copy(tab_hbm.at[tpair], tbuf, sem),
                pltpu.async_copy(rel_hbm.at[rpair], rbuf, sem),
            ]
            for cp in copies:
                cp.wait()

            def outer(bi, sq):
                def rowfn(i, sq):
                    j = bi * 16 + i
                    jv = jnp.full((16,), j, jnp.int32)
                    hm = plsc.load_gather(hoffv, [jv]) > 0
                    tm = plsc.load_gather(toffv, [jv]) > 0
                    rm = plsc.load_gather(roffv, [jv]) > 0
                    hr = hbuf.at[j]
                    tr = tbuf.at[j]
                    rr = rbuf.at[j]
                    p = jnp.zeros((16,), jnp.float32)
                    for g in range(_D // 16):
                        s0 = pl.ds(g * 16, 16)
                        s1 = pl.ds(64 + g * 16, 16)
                        hv = jnp.where(hm, hr[s1], hr[s0])
                        tv = jnp.where(tm, tr[s1], tr[s0])
                        rv = jnp.where(rm, rr[s1], rr[s0])
                        p = p + hv * tv * rv
                        sq = sq + hv * hv + tv * tv + rv * rv
                    pbuf[pl.ds(j * 16, 16)] = p
                    return sq

                sq = lax.fori_loop(0, 16, rowfn, sq)
                acc = jnp.zeros((16,), jnp.float32)
                for j in range(16):
                    acc = acc + plsc.load_gather(
                        pbuf, [(bi * 16 + lane) * 16 + j])
                scores_v[pl.ds(bi * 16, 16)] = acc
                return sq

            sq = lax.fori_loop(0, _CHUNK // 16, outer, sq)
            pltpu.sync_copy(scores_v, scores_out.at[row0])
        for g in range(8):
            sq_v[pl.ds(g * 16, 16)] = jnp.zeros((16,), jnp.float32)
        sq_v[pl.ds(0, 16)] = sq
        pltpu.sync_copy(sq_v, sumsq_out.at[wid])

    return k(table, rel2, h_idx, t_idx, r_idx)


def _tc_finalize(scores2d, sumsq2d):
    nrow = scores2d.shape[0]

    def body(s_ref, q_ref, o_ref):
        s = s_ref[...]
        row = lax.broadcasted_iota(jnp.int32, s.shape, 0)
        y = jnp.where(row < nrow // 2, 1.0, -1.0).astype(jnp.float32)
        z = -s * y
        sp = jnp.maximum(z, 0.0) + jnp.log1p(jnp.exp(-jnp.abs(z)))
        regul = jnp.sum(q_ref[...]) / float(_BT * _D)
        o_ref[0, 0] = jnp.sum(sp) / float(_BT) + _LMBDA * regul

    out = pl.pallas_call(
        body,
        out_shape=jax.ShapeDtypeStruct((1, 1), jnp.float32),
        out_specs=pl.BlockSpec(memory_space=pltpu.SMEM),
    )(scores2d, sumsq2d)
    return out


def kernel(pos_h, pos_r, pos_t, neg_h, neg_r, neg_t, entity_emb, relation_emb):
    h_idx = jnp.concatenate([pos_h, neg_h]).reshape(_BT // _IM, _IM)
    t_idx = jnp.concatenate([pos_t, neg_t]).reshape(_BT // _IM, _IM)
    r_idx = jnp.concatenate([pos_r[:, 0], neg_r[:, 0]]).reshape(_BT // _IM, _IM)
    table = entity_emb.reshape(_E // 2, 128)
    rel2 = relation_emb.reshape(500, 128)
    scores, sumsq = _sc_gather_score(table, rel2, h_idx, t_idx, r_idx)
    out = _tc_finalize(scores, sumsq)
    return out.reshape(())


# speedup vs baseline: 2.0215x; 1.0452x over previous
import functools

import jax
import jax.numpy as jnp
from jax import lax
from jax.experimental import pallas as pl
from jax.experimental.pallas import tpu as pltpu
from jax.experimental.pallas import tpu_sc as plsc

_BT = 32768
_D = 64
_NW = 32
_ROWS_W = _BT // _NW
_CHUNK = 512
_NCHUNK = _ROWS_W // _CHUNK
_IM = 128
_JPC = _CHUNK // _IM
_LMBDA = 0.01


def _sc_gather_score(entity_emb, relation_emb, h_idx, t_idx, r_idx):
    mesh = plsc.VectorSubcoreMesh(core_axis_name="c", subcore_axis_name="s")

    @functools.partial(
        pl.kernel,
        mesh=mesh,
        compiler_params=pltpu.CompilerParams(
            needs_layout_passes=False, use_tc_tiling_on_sc=False),
        out_type=[
            jax.ShapeDtypeStruct((_BT,), jnp.float32),
            jax.ShapeDtypeStruct((_NW, 16), jnp.float32),
        ],
        scratch_types=[
            pltpu.VMEM((_JPC, _IM), jnp.int32),
            pltpu.VMEM((_JPC, _IM), jnp.int32),
            pltpu.VMEM((_JPC, _IM), jnp.int32),
            pltpu.VMEM((_CHUNK, _D), jnp.float32),
            pltpu.VMEM((_CHUNK, _D), jnp.float32),
            pltpu.VMEM((_CHUNK, _D), jnp.float32),
            pltpu.VMEM((_CHUNK,), jnp.float32),
            pltpu.VMEM((256,), jnp.float32),
            pltpu.VMEM((16,), jnp.float32),
            pltpu.SemaphoreType.DMA,
        ],
    )
    def k(ent_hbm, rel_hbm, hidx_hbm, tidx_hbm, ridx_hbm,
          scores_out, sumsq_out,
          hidx_v, tidx_v, ridx_v, hrows, trows, rrows, scores_v, pbuf, sq_v,
          sem):
        wid = lax.axis_index("s") * 2 + lax.axis_index("c")
        lane = lax.iota(jnp.int32, 16)
        sq = jnp.zeros((16,), jnp.float32)
        for c in range(_NCHUNK):
            row0 = wid * (_ROWS_W // _IM) + c * _JPC
            pltpu.sync_copy(hidx_hbm.at[pl.ds(row0, _JPC)], hidx_v)
            pltpu.sync_copy(tidx_hbm.at[pl.ds(row0, _JPC)], tidx_v)
            pltpu.sync_copy(ridx_hbm.at[pl.ds(row0, _JPC)], ridx_v)
            copies = []
            for j in range(_JPC):
                dst = pl.ds(j * _IM, _IM)
                copies.append(pltpu.async_copy(
                    ent_hbm.at[hidx_v.at[j]], hrows.at[dst], sem))
                copies.append(pltpu.async_copy(
                    ent_hbm.at[tidx_v.at[j]], trows.at[dst], sem))
                copies.append(pltpu.async_copy(
                    rel_hbm.at[ridx_v.at[j]], rrows.at[dst], sem))
            for cp in copies:
                cp.wait()

            def outer(bi, sq):
                def rowfn(i, sq):
                    row = bi * 16 + i
                    hr = hrows.at[row]
                    tr = trows.at[row]
                    rr = rrows.at[row]
                    p = jnp.zeros((16,), jnp.float32)
                    for g in range(_D // 16):
                        seg = pl.ds(g * 16, 16)
                        hv = hr[seg]
                        tv = tr[seg]
                        rv = rr[seg]
                        p = p + hv * tv * rv
                        sq = sq + hv * hv + tv * tv + rv * rv
                    pbuf[pl.ds(i * 16, 16)] = p
                    return sq

                sq = lax.fori_loop(0, 16, rowfn, sq)
                acc = jnp.zeros((16,), jnp.float32)
                for j in range(16):
                    acc = acc + plsc.load_gather(pbuf, [lane * 16 + j])
                scores_v[pl.ds(bi * 16, 16)] = acc
                return sq

            sq = lax.fori_loop(0, _CHUNK // 16, outer, sq)
            pltpu.sync_copy(
                scores_v,
                scores_out.at[pl.ds(wid * _ROWS_W + c * _CHUNK, _CHUNK)])
        sq_v[...] = sq
        pltpu.sync_copy(sq_v, sumsq_out.at[wid])

    return k(entity_emb, relation_emb, h_idx, t_idx, r_idx)


def _tc_finalize(scores2d, sumsq2d):
    nrow = scores2d.shape[0]

    def body(s_ref, q_ref, o_ref):
        s = s_ref[...]
        row = lax.broadcasted_iota(jnp.int32, s.shape, 0)
        y = jnp.where(row < nrow // 2, 1.0, -1.0).astype(jnp.float32)
        z = -s * y
        sp = jnp.maximum(z, 0.0) + jnp.log1p(jnp.exp(-jnp.abs(z)))
        regul = jnp.sum(q_ref[...]) / float(_BT * _D)
        o_ref[0, 0] = jnp.sum(sp) / float(_BT) + _LMBDA * regul

    out = pl.pallas_call(
        body,
        out_shape=jax.ShapeDtypeStruct((1, 1), jnp.float32),
        out_specs=pl.BlockSpec(memory_space=pltpu.SMEM),
    )(scores2d, sumsq2d)
    return out


def kernel(pos_h, pos_r, pos_t, neg_h, neg_r, neg_t, entity_emb, relation_emb):
    h_idx = jnp.concatenate([pos_h, neg_h]).reshape(_BT // _IM, _IM)
    t_idx = jnp.concatenate([pos_t, neg_t]).reshape(_BT // _IM, _IM)
    r_idx = jnp.concatenate([pos_r[:, 0], neg_r[:, 0]]).reshape(_BT // _IM, _IM)
    scores, sumsq = _sc_gather_score(
        entity_emb, relation_emb, h_idx, t_idx, r_idx)
    out = _tc_finalize(scores.reshape(_BT // _IM, _IM),
                       sumsq.reshape(_NW * 16 // _IM, _IM))
    return out.reshape(())
